# Initial kernel scaffold; baseline (speedup 1.0000x reference)
import jax
import jax.numpy as jnp
from jax.experimental import pallas as pl


def kernel(Prob):
    raise NotImplementedError("write your pallas kernel here")


# speedup vs baseline: 7.7517x; 7.7517x over previous
import jax
import jax.numpy as jnp
from jax.experimental import pallas as pl

_TOPN = 64
_INF_BITS = 0x7F800000


def _topk_mask_kernel(x_ref, o_ref):
    x = x_ref[...]
    B, N = x.shape
    b = jax.lax.bitcast_convert_type(x, jnp.int32)

    lo = jnp.zeros((B, 1), jnp.int32)
    hi = jnp.full((B, 1), _INF_BITS, jnp.int32)

    def val_body(_, lohi):
        lo, hi = lohi
        mid = lo + ((hi - lo) >> 1)
        cnt = jnp.sum((b >= mid).astype(jnp.int32), axis=1, keepdims=True)
        ge = cnt >= _TOPN
        return jnp.where(ge, mid, lo), jnp.where(ge, hi, mid)

    lo, hi = jax.lax.fori_loop(0, 31, val_body, (lo, hi))
    thresh = lo

    gt = b > thresh
    eq = b == thresh
    cnt_gt = jnp.sum(gt.astype(jnp.int32), axis=1, keepdims=True)
    need = _TOPN - cnt_gt

    iota = jax.lax.broadcasted_iota(jnp.int32, (B, N), 1)
    lo2 = jnp.full((B, 1), -1, jnp.int32)
    hi2 = jnp.full((B, 1), N, jnp.int32)

    def idx_body(_, lohi):
        lo, hi = lohi
        mid = lo + ((hi - lo) >> 1)
        cnt = jnp.sum((eq & (iota < mid)).astype(jnp.int32), axis=1,
                      keepdims=True)
        ok = cnt >= need
        return jnp.where(ok, lo, mid), jnp.where(ok, mid, hi)

    lo2, hi2 = jax.lax.fori_loop(0, 16, idx_body, (lo2, hi2))
    cut = hi2

    keep = gt | (eq & (iota < cut))
    pm = jnp.where(keep, x, 0.0)
    s = jnp.sum(pm, axis=1, keepdims=True)
    o_ref[...] = pm / s


def kernel(Prob):
    return pl.pallas_call(
        _topk_mask_kernel,
        out_shape=jax.ShapeDtypeStruct(Prob.shape, Prob.dtype),
    )(Prob)


# speedup vs baseline: 15.5099x; 2.0008x over previous
import jax
import jax.numpy as jnp
from jax.experimental import pallas as pl

_TOPN = 64


def _count_ge(b, t):
    return jnp.sum((b >= t).astype(jnp.int32), axis=1, keepdims=True)


def _bisect_threshold(b, lo, hi, steps):
    def body(carry):
        lo, hi, _ = carry
        mid = lo + ((hi - lo) >> 1)
        ge = _count_ge(b, mid) >= _TOPN
        lo = jnp.where(ge, mid, lo)
        hi = jnp.where(ge, hi, mid)
        return lo, hi, jnp.any(hi - lo > 1)

    def cond(carry):
        return carry[2]

    lo, hi, _ = jax.lax.while_loop(
        cond, lambda c: body(c), (lo, hi, jnp.bool_(True)))
    del steps
    return lo


def _topk_mask_kernel(x_ref, o_ref):
    x = x_ref[...]
    B, N = x.shape
    b = jax.lax.bitcast_convert_type(x, jnp.int32)

    g = b
    w = N
    while w > 128:
        w //= 2
        g = jnp.maximum(g[:, :w], g[:, w:])
    gmax = jnp.max(g, axis=1, keepdims=True)

    lo_s = jnp.zeros((B, 1), jnp.int32)
    hi_s = gmax + 1
    lo_s = _bisect_threshold(g, lo_s, hi_s, 31)

    thresh = _bisect_threshold(b, lo_s, gmax + 1, 31)

    gt = b > thresh
    eq = b == thresh
    cnt_gt = jnp.sum(gt.astype(jnp.int32), axis=1, keepdims=True)
    cnt_eq = jnp.sum(eq.astype(jnp.int32), axis=1, keepdims=True)
    need = _TOPN - cnt_gt

    iota = jax.lax.broadcasted_iota(jnp.int32, (B, N), 1)

    def tie_cut(_):
        lo2 = jnp.full((B, 1), -1, jnp.int32)
        hi2 = jnp.full((B, 1), N, jnp.int32)

        def body(_, lohi):
            lo, hi = lohi
            mid = lo + ((hi - lo) >> 1)
            cnt = jnp.sum((eq & (iota < mid)).astype(jnp.int32), axis=1,
                          keepdims=True)
            ok = cnt >= need
            return jnp.where(ok, lo, mid), jnp.where(ok, mid, hi)

        _, hi2 = jax.lax.fori_loop(0, 16, body, (lo2, hi2))
        return hi2

    no_ties = jnp.all(cnt_eq == need)
    cut = jax.lax.cond(no_ties, lambda _: jnp.full((B, 1), N, jnp.int32),
                       tie_cut, operand=None)

    keep = gt | (eq & (iota < cut))
    pm = jnp.where(keep, x, 0.0)
    s = jnp.sum(pm, axis=1, keepdims=True)
    o_ref[...] = pm / s


def kernel(Prob):
    return pl.pallas_call(
        _topk_mask_kernel,
        out_shape=jax.ShapeDtypeStruct(Prob.shape, Prob.dtype),
    )(Prob)
